# baseline (speedup 1.0000x reference)
import functools

import jax
import jax.numpy as jnp
from jax import lax
from jax.experimental import pallas as pl
from jax.experimental.pallas import tpu as pltpu
from jax.experimental.pallas import tpu_sc as plsc

NUM_ITEM = 100000
START_USER = NUM_ITEM
BATCH = 16384
EMBED_DIM = 128

NC = 2
NS = 16
NW = NC * NS
B_PER_W = BATCH // NW
CHUNK = 128
NCHUNK = B_PER_W // CHUNK
LANES = 16


@functools.partial(
    pl.kernel,
    out_type=jax.ShapeDtypeStruct((BATCH, EMBED_DIM), jnp.float32),
    mesh=plsc.VectorSubcoreMesh(core_axis_name="c", subcore_axis_name="s"),
    scratch_types=[
        pltpu.VMEM((B_PER_W,), jnp.int32),
        pltpu.VMEM((B_PER_W, EMBED_DIM), jnp.float32),
    ] + [pltpu.SemaphoreType.DMA] * (NCHUNK + 1),
)
def _sc_gather(table_hbm, idx_hbm, out_hbm, idx_v, rows_v, *sems):
    gsems, osem = sems[:NCHUNK], sems[NCHUNK]
    wid = lax.axis_index("s") * NC + lax.axis_index("c")
    base = wid * B_PER_W
    pltpu.sync_copy(idx_hbm.at[pl.ds(base, B_PER_W)], idx_v)
    user_block = table_hbm.at[pl.ds(START_USER, NUM_ITEM + 1)]

    def gather(j):
        return pltpu.async_copy(
            user_block.at[idx_v.at[pl.ds(j * CHUNK, CHUNK)]],
            rows_v.at[pl.ds(j * CHUNK, CHUNK)],
            gsems[j],
        )

    def write(j):
        return pltpu.async_copy(
            rows_v.at[pl.ds(j * CHUNK, CHUNK)],
            out_hbm.at[pl.ds(base + j * CHUNK, CHUNK)],
            osem,
        )

    gathers = [gather(0)]
    writes = []
    for j in range(NCHUNK):
        gathers[j].wait()
        writes.append(write(j))
        if j + 1 < NCHUNK:
            gathers.append(gather(j + 1))
    for cp in writes:
        cp.wait()


def kernel(emb_weight, batch):
    return _sc_gather(emb_weight, batch.astype(jnp.int32))


# speedup vs baseline: 1.0739x; 1.0739x over previous
import functools

import jax
import jax.numpy as jnp
from jax import lax
from jax.experimental import pallas as pl
from jax.experimental.pallas import tpu as pltpu
from jax.experimental.pallas import tpu_sc as plsc

NUM_ITEM = 100000
START_USER = NUM_ITEM
BATCH = 16384
EMBED_DIM = 128

NC = 2
NS = 16
NW = NC * NS
B_PER_W = BATCH // NW
CHUNK = 128
NCHUNK = B_PER_W // CHUNK
LANES = 16


@functools.partial(
    pl.kernel,
    out_type=jax.ShapeDtypeStruct((BATCH, EMBED_DIM), jnp.float32),
    mesh=plsc.VectorSubcoreMesh(core_axis_name="c", subcore_axis_name="s"),
    scratch_types=[
        pltpu.VMEM((B_PER_W,), jnp.int32),
        pltpu.VMEM((B_PER_W, EMBED_DIM), jnp.float32),
    ] + [pltpu.SemaphoreType.DMA] * (NCHUNK + 1),
)
def _sc_gather(table_hbm, idx_hbm, out_hbm, idx_v, rows_v, *sems):
    gsems, osem = sems[:NCHUNK], sems[NCHUNK]
    wid = lax.axis_index("s") * NC + lax.axis_index("c")
    base = wid * B_PER_W
    pltpu.sync_copy(idx_hbm.at[pl.ds(base, B_PER_W)], idx_v)
    user_block = table_hbm.at[pl.ds(START_USER, NUM_ITEM + 1)]
    gathers = [
        pltpu.async_copy(
            user_block.at[idx_v.at[pl.ds(j * CHUNK, CHUNK)]],
            rows_v.at[pl.ds(j * CHUNK, CHUNK)],
            gsems[j],
        )
        for j in range(NCHUNK)
    ]
    writes = []
    for j in range(NCHUNK):
        gathers[j].wait()
        writes.append(
            pltpu.async_copy(
                rows_v.at[pl.ds(j * CHUNK, CHUNK)],
                out_hbm.at[pl.ds(base + j * CHUNK, CHUNK)],
                osem,
            )
        )
    for cp in writes:
        cp.wait()


def kernel(emb_weight, batch):
    return _sc_gather(emb_weight, batch.astype(jnp.int32))
